# baseline (speedup 1.0000x reference)
import numpy as np
import jax
import jax.numpy as jnp
from jax import lax
from jax.experimental import pallas as pl
from jax.experimental.pallas import tpu as pltpu

_KTAPS = {(0, 0): (0,), (0, 1): (1, 2), (1, 0): (0, 1), (1, 1): (2,)}

_PAD = 192


def _upsample_matrix(H, W):
    HW = H * W
    U = np.zeros((HW, 4 * HW), np.float32)
    i = np.arange(H)[:, None]
    j = np.arange(W)[None, :]
    src = (i * W + j).ravel()
    for a in (0, 1):
        for b in (0, 1):
            dst = ((2 * i + a) * 2 * W + 2 * j + b).ravel()
            U[src, dst] = 1.0
    return jnp.asarray(U, jnp.bfloat16)


def _dot0(a, b):
    return lax.dot_general(a, b, (((0,), (0,)), ((), ())),
                           preferred_element_type=jnp.float32)


def _make_kernel(H, W, Cin, Cout, Cint):
    HW = H * W
    HW4 = 4 * HW
    W2 = 2 * W
    CH = min(512, HW4)

    def body(x1_ref, x2_ref, up_ref, wc_ref, bc_ref, wl1_ref, bl1_ref,
             wl2_ref, bl2_ref, wg1_ref, bg1_ref, wg2_ref, bg2_ref, o_ref,
             xp_scr, y_scr):
        tap_classes = {}
        for p in (-1, 0, 1):
            for q in (-1, 0, 1):
                cls = [(a, b) for a in (0, 1) for b in (0, 1)
                       if 0 <= p + 1 - a <= 1 and 0 <= q + 1 - b <= 1]
                tap_classes[(p, q)] = cls
        wtap = {}
        for (p, q), cls in tap_classes.items():
            mats = []
            for (a, b) in cls:
                u, vv = p + 1 - a, q + 1 - b
                acc = None
                for kh in _KTAPS[(a, u)]:
                    for kw in _KTAPS[(b, vv)]:
                        t = wc_ref[kh * 3 + kw]
                        acc = t if acc is None else acc + t
                mats.append(acc.astype(jnp.bfloat16))
            wtap[(p, q)] = jnp.concatenate(mats, axis=1)

        x1b = x1_ref[0].astype(jnp.bfloat16)
        xp_scr[:, :_PAD] = jnp.zeros((Cin, _PAD), jnp.bfloat16)
        xp_scr[:, _PAD + HW4:] = jnp.zeros((Cin, _PAD), jnp.bfloat16)
        for c0 in range(0, HW4, CH):
            seg = jnp.dot(x1b, up_ref[:, c0:c0 + CH],
                          preferred_element_type=jnp.float32)
            xp_scr[:, _PAD + c0:_PAD + c0 + CH] = seg.astype(jnp.bfloat16)

        bc = bc_ref[...]
        ysum = jnp.zeros((Cout, 1), jnp.float32)

        for c0 in range(0, HW4, CH):
            t = c0 + lax.broadcasted_iota(jnp.int32, (1, CH), 1)
            j_idx = (t % W2) // 2
            a_par = (t // W2) % 2
            b_par = t % 2

            taps = {}
            for p in (-1, 0, 1):
                for q in (-1, 0, 1):
                    s = _PAD + c0 + p * 2 * W2 + 2 * q
                    v = xp_scr[:, s:s + CH]
                    if q == -1:
                        v = jnp.where(j_idx >= 1, v, jnp.bfloat16(0))
                    elif q == 1:
                        v = jnp.where(j_idx < W - 1, v, jnp.bfloat16(0))
                    taps[(p, q)] = v

            acc = {}
            for (p, q), cls in tap_classes.items():
                d = _dot0(wtap[(p, q)], taps[(p, q)])
                for k, ab in enumerate(cls):
                    part = d[k * Cout:(k + 1) * Cout]
                    acc[ab] = part if ab not in acc else acc[ab] + part

            ya = jnp.where(b_par == 1, acc[(0, 1)], acc[(0, 0)])
            yb = jnp.where(b_par == 1, acc[(1, 1)], acc[(1, 0)])
            y = jnp.maximum(jnp.where(a_par == 1, yb, ya) + bc, 0.0)
            ysum = ysum + jnp.sum(y, axis=1, keepdims=True)
            y_scr[:, c0:c0 + CH] = y

        x2sum = jnp.zeros((Cout, 1), jnp.float32)
        for c0 in range(0, HW4, CH):
            x2sum = x2sum + jnp.sum(x2_ref[0, :, c0:c0 + CH], axis=1,
                                    keepdims=True)
        m = (ysum + x2sum) * (1.0 / HW4)
        g1 = jnp.maximum(_dot0(wg1_ref[...], m) + bg1_ref[...], 0.0)
        g2 = _dot0(wg2_ref[...], g1) + bg2_ref[...]

        wl1b = wl1_ref[...].astype(jnp.bfloat16)
        wl2b = wl2_ref[...].astype(jnp.bfloat16)
        for c0 in range(0, HW4, CH):
            y = y_scr[:, c0:c0 + CH]
            x2 = x2_ref[0, :, c0:c0 + CH]
            xa = (y + x2).astype(jnp.bfloat16)
            l1 = jnp.maximum(_dot0(wl1b, xa) + bl1_ref[...], 0.0)
            l2 = _dot0(wl2b, l1.astype(jnp.bfloat16)) + bl2_ref[...]
            wei = jax.nn.sigmoid(l2 + g2)
            o_ref[0, :, c0:c0 + CH] = 2.0 * y * wei + 2.0 * x2 * (1.0 - wei)

    return body


def kernel(x1_nchw, x2_nchw, wconv_f, bconv_f, wl1_f, bl1_f, wl2_f, bl2_f,
           wg1_f, bg1_f, wg2_f, bg2_f):
    N, Cin, H, W = x1_nchw.shape
    Cout = x2_nchw.shape[1]
    Cint = wl1_f.shape[1]
    HW = H * W
    HW4 = 4 * HW

    x1f = x1_nchw.reshape(N, Cin, HW)
    x2f = x2_nchw.reshape(N, Cout, HW4)
    wc9 = wconv_f.reshape(9, Cin, Cout)
    up = _upsample_matrix(H, W)

    bc = bconv_f.reshape(Cout, 1)
    bl1 = bl1_f.reshape(Cint, 1)
    bl2 = bl2_f.reshape(Cout, 1)
    bg1 = bg1_f.reshape(Cint, 1)
    bg2 = bg2_f.reshape(Cout, 1)

    weights = (up, wc9, bc, wl1_f, bl1, wl2_f, bl2, wg1_f, bg1, wg2_f, bg2)

    def full(arr):
        rank = arr.ndim
        return pl.BlockSpec(arr.shape, lambda n, _r=rank: (0,) * _r)

    out = pl.pallas_call(
        _make_kernel(H, W, Cin, Cout, Cint),
        out_shape=jax.ShapeDtypeStruct((N, Cout, HW4), jnp.float32),
        grid=(N,),
        in_specs=[
            pl.BlockSpec((1, Cin, HW), lambda n: (n, 0, 0)),
            pl.BlockSpec((1, Cout, HW4), lambda n: (n, 0, 0)),
        ] + [full(w) for w in weights],
        out_specs=pl.BlockSpec((1, Cout, HW4), lambda n: (n, 0, 0)),
        scratch_shapes=[
            pltpu.VMEM((Cin, HW4 + 2 * _PAD), jnp.bfloat16),
            pltpu.VMEM((Cout, HW4), jnp.float32),
        ],
        compiler_params=pltpu.CompilerParams(
            dimension_semantics=("parallel",)),
    )(x1f, x2f, *weights)

    return out.reshape(N, Cout, 2 * H, 2 * W)


# speedup vs baseline: 1.0322x; 1.0322x over previous
import numpy as np
import jax
import jax.numpy as jnp
from jax import lax
from jax.experimental import pallas as pl
from jax.experimental.pallas import tpu as pltpu

_KTAPS = {(0, 0): (0,), (0, 1): (1, 2), (1, 0): (0, 1), (1, 1): (2,)}

_PAD = 192


def _upsample_matrix(H, W):
    HW = H * W
    U = np.zeros((HW, 4 * HW), np.float32)
    i = np.arange(H)[:, None]
    j = np.arange(W)[None, :]
    src = (i * W + j).ravel()
    for a in (0, 1):
        for b in (0, 1):
            dst = ((2 * i + a) * 2 * W + 2 * j + b).ravel()
            U[src, dst] = 1.0
    return jnp.asarray(U, jnp.bfloat16)


def _dot0(a, b):
    return lax.dot_general(a, b, (((0,), (0,)), ((), ())),
                           preferred_element_type=jnp.float32)


def _make_kernel(H, W, Cin, Cout, Cint):
    HW = H * W
    HW4 = 4 * HW
    W2 = 2 * W
    CH = min(2048, HW4)

    def body(x1_ref, x2_ref, up_ref, wc_ref, bc_ref, wl1_ref, bl1_ref,
             wl2_ref, bl2_ref, wg1_ref, bg1_ref, wg2_ref, bg2_ref, o_ref,
             xp_scr, y_scr):
        tap_classes = {}
        for p in (-1, 0, 1):
            for q in (-1, 0, 1):
                cls = [(a, b) for a in (0, 1) for b in (0, 1)
                       if 0 <= p + 1 - a <= 1 and 0 <= q + 1 - b <= 1]
                tap_classes[(p, q)] = cls
        wtap = {}
        for (p, q), cls in tap_classes.items():
            mats = []
            for (a, b) in cls:
                u, vv = p + 1 - a, q + 1 - b
                acc = None
                for kh in _KTAPS[(a, u)]:
                    for kw in _KTAPS[(b, vv)]:
                        t = wc_ref[kh * 3 + kw]
                        acc = t if acc is None else acc + t
                mats.append(acc.astype(jnp.bfloat16))
            wtap[(p, q)] = jnp.concatenate(mats, axis=1)

        x1b = x1_ref[0].astype(jnp.bfloat16)
        xp_scr[:, :_PAD] = jnp.zeros((Cin, _PAD), jnp.bfloat16)
        xp_scr[:, _PAD + HW4:] = jnp.zeros((Cin, _PAD), jnp.bfloat16)
        for c0 in range(0, HW4, CH):
            seg = jnp.dot(x1b, up_ref[:, c0:c0 + CH],
                          preferred_element_type=jnp.float32)
            xp_scr[:, _PAD + c0:_PAD + c0 + CH] = seg.astype(jnp.bfloat16)

        bc = bc_ref[...]
        ysum = jnp.zeros((Cout, 1), jnp.float32)

        for c0 in range(0, HW4, CH):
            t = c0 + lax.broadcasted_iota(jnp.int32, (1, CH), 1)
            j_idx = (t % W2) // 2
            a_par = (t // W2) % 2
            b_par = t % 2

            taps = {}
            for p in (-1, 0, 1):
                for q in (-1, 0, 1):
                    s = _PAD + c0 + p * 2 * W2 + 2 * q
                    v = xp_scr[:, s:s + CH]
                    if q == -1:
                        v = jnp.where(j_idx >= 1, v, jnp.bfloat16(0))
                    elif q == 1:
                        v = jnp.where(j_idx < W - 1, v, jnp.bfloat16(0))
                    taps[(p, q)] = v

            acc = {}
            for (p, q), cls in tap_classes.items():
                d = _dot0(wtap[(p, q)], taps[(p, q)])
                for k, ab in enumerate(cls):
                    part = d[k * Cout:(k + 1) * Cout]
                    acc[ab] = part if ab not in acc else acc[ab] + part

            ya = jnp.where(b_par == 1, acc[(0, 1)], acc[(0, 0)])
            yb = jnp.where(b_par == 1, acc[(1, 1)], acc[(1, 0)])
            y = jnp.maximum(jnp.where(a_par == 1, yb, ya) + bc, 0.0)
            ysum = ysum + jnp.sum(y, axis=1, keepdims=True)
            y_scr[:, c0:c0 + CH] = y

        x2sum = jnp.zeros((Cout, 1), jnp.float32)
        for c0 in range(0, HW4, CH):
            x2sum = x2sum + jnp.sum(x2_ref[0, :, c0:c0 + CH], axis=1,
                                    keepdims=True)
        m = (ysum + x2sum) * (1.0 / HW4)
        g1 = jnp.maximum(_dot0(wg1_ref[...], m) + bg1_ref[...], 0.0)
        g2 = _dot0(wg2_ref[...], g1) + bg2_ref[...]

        wl1b = wl1_ref[...].astype(jnp.bfloat16)
        wl2b = wl2_ref[...].astype(jnp.bfloat16)
        for c0 in range(0, HW4, CH):
            y = y_scr[:, c0:c0 + CH]
            x2 = x2_ref[0, :, c0:c0 + CH]
            xa = (y + x2).astype(jnp.bfloat16)
            l1 = jnp.maximum(_dot0(wl1b, xa) + bl1_ref[...], 0.0)
            l2 = _dot0(wl2b, l1.astype(jnp.bfloat16)) + bl2_ref[...]
            wei = jax.nn.sigmoid(l2 + g2)
            o_ref[0, :, c0:c0 + CH] = 2.0 * y * wei + 2.0 * x2 * (1.0 - wei)

    return body


def kernel(x1_nchw, x2_nchw, wconv_f, bconv_f, wl1_f, bl1_f, wl2_f, bl2_f,
           wg1_f, bg1_f, wg2_f, bg2_f):
    N, Cin, H, W = x1_nchw.shape
    Cout = x2_nchw.shape[1]
    Cint = wl1_f.shape[1]
    HW = H * W
    HW4 = 4 * HW

    x1f = x1_nchw.reshape(N, Cin, HW)
    x2f = x2_nchw.reshape(N, Cout, HW4)
    wc9 = wconv_f.reshape(9, Cin, Cout)
    up = _upsample_matrix(H, W)

    bc = bconv_f.reshape(Cout, 1)
    bl1 = bl1_f.reshape(Cint, 1)
    bl2 = bl2_f.reshape(Cout, 1)
    bg1 = bg1_f.reshape(Cint, 1)
    bg2 = bg2_f.reshape(Cout, 1)

    weights = (up, wc9, bc, wl1_f, bl1, wl2_f, bl2, wg1_f, bg1, wg2_f, bg2)

    def full(arr):
        rank = arr.ndim
        return pl.BlockSpec(arr.shape, lambda n, _r=rank: (0,) * _r)

    out = pl.pallas_call(
        _make_kernel(H, W, Cin, Cout, Cint),
        out_shape=jax.ShapeDtypeStruct((N, Cout, HW4), jnp.float32),
        grid=(N,),
        in_specs=[
            pl.BlockSpec((1, Cin, HW), lambda n: (n, 0, 0)),
            pl.BlockSpec((1, Cout, HW4), lambda n: (n, 0, 0)),
        ] + [full(w) for w in weights],
        out_specs=pl.BlockSpec((1, Cout, HW4), lambda n: (n, 0, 0)),
        scratch_shapes=[
            pltpu.VMEM((Cin, HW4 + 2 * _PAD), jnp.bfloat16),
            pltpu.VMEM((Cout, HW4), jnp.float32),
        ],
        compiler_params=pltpu.CompilerParams(
            dimension_semantics=("parallel",)),
    )(x1f, x2f, *weights)

    return out.reshape(N, Cout, 2 * H, 2 * W)
